# baseline (speedup 1.0000x reference)
import math

import jax
import jax.numpy as jnp
from jax.experimental import pallas as pl
from jax.experimental.pallas import tpu as pltpu

_B, _S, _DIN, _DOUT = 32, 64, 2048, 2048
_BASE_LR = 0.01
_SCALAR_SCALER = math.sqrt(_DIN)
_LR_SCALE = _BASE_LR * math.sqrt(max(_DIN, _DOUT)) * math.sqrt(1.0 / _DIN)

_BO = 512


def _body(x_ref, log_lr_ref, state_ref, base_ref, o_ref, lr_scr):
    b = pl.program_id(1)

    @pl.when(b == 0)
    def _():
        lr_scr[...] = _LR_SCALE * jnp.exp(log_lr_ref[...] * _SCALAR_SCALER)

    w = lr_scr[...] * state_ref[0] + base_ref[...]
    o_ref[0] = jax.lax.dot_general(
        x_ref[0], w, (((1,), (1,)), ((), ())),
        preferred_element_type=jnp.float32)


def _call(x, log_lr, state, base_w, interpret=False):
    n_ob = _DOUT // _BO
    return pl.pallas_call(
        _body,
        out_shape=jax.ShapeDtypeStruct((_B, _S, _DOUT), jnp.float32),
        grid=(n_ob, _B),
        in_specs=[
            pl.BlockSpec((1, _S, _DIN), lambda ob, b: (b, 0, 0)),
            pl.BlockSpec((_BO, _DIN), lambda ob, b: (ob, 0)),
            pl.BlockSpec((1, _BO, _DIN), lambda ob, b: (b, ob, 0)),
            pl.BlockSpec((_BO, _DIN), lambda ob, b: (ob, 0)),
        ],
        out_specs=pl.BlockSpec((1, _S, _BO), lambda ob, b: (b, 0, ob)),
        scratch_shapes=[pltpu.VMEM((_BO, _DIN), jnp.float32)],
        compiler_params=pltpu.CompilerParams(
            dimension_semantics=("parallel", "arbitrary"),
            vmem_limit_bytes=50 * 1024 * 1024,
        ),
        name="ittt_linear",
        interpret=interpret,
    )(x, log_lr, state, base_w)


def kernel(x, log_lr, state, momentum, base_w):
    del momentum
    return _call(x, log_lr, state, base_w)


# speedup vs baseline: 1.3281x; 1.3281x over previous
import math

import jax
import jax.numpy as jnp
from jax.experimental import pallas as pl
from jax.experimental.pallas import tpu as pltpu

_B, _S, _DIN, _DOUT = 32, 64, 2048, 2048
_BASE_LR = 0.01
_SCALAR_SCALER = math.sqrt(_DIN)
_LR_SCALE = _BASE_LR * math.sqrt(max(_DIN, _DOUT)) * math.sqrt(1.0 / _DIN)

_BO = 1024


def _body(x_ref, log_lr_ref, state_ref, base_ref, o_ref, lr_scr):
    b = pl.program_id(1)

    @pl.when(b == 0)
    def _():
        lr_scr[...] = (
            _LR_SCALE * jnp.exp(log_lr_ref[...] * _SCALAR_SCALER)
        ).astype(jnp.bfloat16)

    w = lr_scr[...].astype(jnp.float32) * state_ref[0] + base_ref[...]
    o_ref[0] = jax.lax.dot_general(
        x_ref[0], w, (((1,), (1,)), ((), ())),
        preferred_element_type=jnp.float32)


def _call(x, log_lr, state, base_w, interpret=False):
    n_ob = _DOUT // _BO
    return pl.pallas_call(
        _body,
        out_shape=jax.ShapeDtypeStruct((_B, _S, _DOUT), jnp.float32),
        grid=(n_ob, _B),
        in_specs=[
            pl.BlockSpec((1, _S, _DIN), lambda ob, b: (b, 0, 0)),
            pl.BlockSpec((_BO, _DIN), lambda ob, b: (ob, 0)),
            pl.BlockSpec((1, _BO, _DIN), lambda ob, b: (b, ob, 0)),
            pl.BlockSpec((_BO, _DIN), lambda ob, b: (ob, 0)),
        ],
        out_specs=pl.BlockSpec((1, _S, _BO), lambda ob, b: (b, 0, ob)),
        scratch_shapes=[pltpu.VMEM((_BO, _DIN), jnp.bfloat16)],
        compiler_params=pltpu.CompilerParams(
            dimension_semantics=("parallel", "arbitrary"),
            vmem_limit_bytes=56 * 1024 * 1024,
        ),
        name="ittt_linear",
        interpret=interpret,
    )(x, log_lr, state, base_w)


def kernel(x, log_lr, state, momentum, base_w):
    del momentum
    return _call(x, log_lr, state, base_w)
